# baseline (speedup 1.0000x reference)
import functools

import jax
import jax.numpy as jnp
from jax import lax
from jax.experimental import pallas as pl
from jax.experimental.pallas import tpu as pltpu
from jax.experimental.pallas import tpu_sc as plsc

N = 10000
E = 320000
D1 = 128
D2 = 64
NC, NS = 2, 16
NW = NC * NS
EPW = E // NW
K = 80
NG = K // 16
NCH = EPW // K
ACC_N = 10240
WR = ACC_N // 128
BLK = 1024
NBLK = ACC_N // BLK
EPS = 1e-16

_GDN = lax.GatherDimensionNumbers(
    offset_dims=(), collapsed_slice_dims=(0,), start_index_map=(0,)
)


def _lane_perm(v, idx):
    return lax.gather(v, idx[:, None], _GDN, slice_sizes=(1,),
                      mode=lax.GatherScatterMode.PROMISE_IN_BOUNDS)


def _lane_sum_bcast(v):
    t = plsc.cumsum(v)
    return _lane_perm(t, jnp.full((16,), 15, jnp.int32))


def _make_edge_kernel(D):
    cgn = D // 16
    rows_per_sub = ACC_N // NS
    mesh = plsc.VectorSubcoreMesh(
        core_axis_name="c", subcore_axis_name="s", num_cores=NC, num_subcores=NS
    )

    @functools.partial(
        pl.kernel,
        out_type=[
            jax.ShapeDtypeStruct((NC, ACC_N, D), jnp.float32),
            jax.ShapeDtypeStruct((NC, WR, 128), jnp.float32),
        ],
        mesh=mesh,
        scratch_types=[
            pltpu.VMEM((K,), jnp.int32),
            pltpu.VMEM((K,), jnp.int32),
            pltpu.VMEM((K,), jnp.int32),
            pltpu.VMEM((K,), jnp.int32),
            pltpu.VMEM((K,), jnp.int32),
            pltpu.VMEM((K,), jnp.int32),
            pltpu.VMEM((K, D), jnp.float32),
            pltpu.VMEM((K, D), jnp.float32),
            pltpu.VMEM((K, D), jnp.float32),
            pltpu.VMEM((K * 16,), jnp.float32),
            pltpu.VMEM((K, 128), jnp.float32),
            pltpu.VMEM((D,), jnp.float32),
            pltpu.VMEM_SHARED((ACC_N, D), jnp.float32),
            pltpu.VMEM_SHARED((WR, 128), jnp.float32),
            pltpu.SemaphoreType.DMA,
            pltpu.SemaphoreType.DMA,
            pltpu.SemaphoreType.DMA,
            pltpu.SemaphoreType.DMA,
            pltpu.SemaphoreType.DMA,
            pltpu.SemaphoreType.DMA,
        ],
        compiler_params=pltpu.CompilerParams(needs_layout_passes=False),
    )
    def edge_kernel(tabl_hbm, tabr_hbm, src_hbm, dst_hbm, att_hbm,
                    out_feat, out_w,
                    src_v0, dst_v0, ddiv_v0, src_v1, dst_v1, ddiv_v1,
                    rows_l, rows_r, packed, wbuf,
                    wrows, att_v, acc_sh, accw_sh,
                    sem1, sem2, sems1, sems2, semidx1, semidx2):
        cid = lax.axis_index("c")
        sid = lax.axis_index("s")
        wid = sid * NC + cid
        zero16 = jnp.zeros((16,), jnp.float32)
        iota16 = lax.iota(jnp.int32, 16)

        def zp(i, carry):
            for j in range(cgn):
                packed[i, pl.ds(j * 16, 16)] = zero16
            for j in range(8):
                wrows[i, pl.ds(j * 16, 16)] = zero16
            return carry
        lax.fori_loop(0, K, zp, 0)

        for t in range(rows_per_sub // K):
            pltpu.sync_copy(
                packed, acc_sh.at[pl.ds(sid * rows_per_sub + t * K, K)]
            )

        @pl.when(sid == 0)
        def _zero_esum():
            pltpu.sync_copy(wrows, accw_sh)

        pltpu.sync_copy(att_hbm, att_v)
        att_regs = [att_v[pl.ds(j * 16, 16)] for j in range(cgn)]
        plsc.subcore_barrier()

        idx_bufs = [(src_v0, dst_v0, ddiv_v0), (src_v1, dst_v1, ddiv_v1)]

        def do_chunk(k, cur, other, first):
            srcb, dstb, ddivb = cur
            psrc, pdst, pddiv = other
            if not first:
                pltpu.make_async_copy(packed, acc_sh.at[pdst], sems1).wait()
                pltpu.make_async_copy(wrows, accw_sh.at[pddiv], sems2).wait()
                for g in range(NG):
                    eids = g * 16 + iota16
                    dsts = pdst[pl.ds(g * 16, 16)]
                    dmod = lax.bitwise_and(dsts, 127)
                    plsc.store_scatter(wrows, [eids, dmod], zero16)
            nbase = wid * EPW + jnp.minimum(k + 1, NCH - 1) * K
            pltpu.async_copy(src_hbm.at[pl.ds(nbase, K)], psrc, semidx1)
            pltpu.async_copy(dst_hbm.at[pl.ds(nbase, K)], pdst, semidx2)
            pltpu.make_async_copy(tabl_hbm.at[srcb], rows_l, sem1).wait()
            pltpu.make_async_copy(tabr_hbm.at[dstb], rows_r, sem2).wait()

            @plsc.parallel_loop(0, K, unroll=4)
            def edge_body(e):
                ls = []
                terms = []
                for j in range(cgn):
                    l = rows_l[e, pl.ds(j * 16, 16)]
                    r = rows_r[e, pl.ds(j * 16, 16)]
                    ls.append(l)
                    s = l + r
                    lr = jnp.maximum(s, 0.2 * s)
                    terms.append(lr * att_regs[j])
                while len(terms) > 1:
                    terms = [a + b for a, b in zip(terms[::2], terms[1::2])]
                w = jnp.exp(_lane_sum_bcast(terms[0]))
                for j in range(cgn):
                    packed[e, pl.ds(j * 16, 16)] = ls[j] * w
                wbuf[pl.ds(e * 16, 16)] = w

            for g in range(NG):
                eids = g * 16 + iota16
                dsts = dstb[pl.ds(g * 16, 16)]
                dmod = lax.bitwise_and(dsts, 127)
                ddiv = lax.shift_right_logical(dsts, 7)
                ddivb[pl.ds(g * 16, 16)] = ddiv
                wv = plsc.load_gather(wbuf, [eids * 16])
                plsc.store_scatter(wrows, [eids, dmod], wv)

            pltpu.async_copy(packed, acc_sh.at[dstb], sems1, add=True)
            pltpu.async_copy(wrows, accw_sh.at[ddivb], sems2, add=True)
            pltpu.make_async_copy(
                src_hbm.at[pl.ds(0, K)], psrc, semidx1).wait()
            pltpu.make_async_copy(
                dst_hbm.at[pl.ds(0, K)], pdst, semidx2).wait()
            pltpu.async_copy(tabl_hbm.at[psrc], rows_l, sem1)
            pltpu.async_copy(tabr_hbm.at[pdst], rows_r, sem2)

        base0 = wid * EPW
        pltpu.sync_copy(src_hbm.at[pl.ds(base0, K)], src_v0)
        pltpu.sync_copy(dst_hbm.at[pl.ds(base0, K)], dst_v0)
        pltpu.async_copy(tabl_hbm.at[src_v0], rows_l, sem1)
        pltpu.async_copy(tabr_hbm.at[dst_v0], rows_r, sem2)
        do_chunk(0, idx_bufs[0], idx_bufs[1], True)

        def pair_body(i, carry):
            do_chunk(1 + 2 * i, idx_bufs[1], idx_bufs[0], False)
            do_chunk(2 + 2 * i, idx_bufs[0], idx_bufs[1], False)
            return carry
        lax.fori_loop(0, (NCH - 1) // 2, pair_body, 0)

        pltpu.make_async_copy(packed, acc_sh.at[dst_v0], sems1).wait()
        pltpu.make_async_copy(wrows, accw_sh.at[ddiv_v0], sems2).wait()
        pltpu.make_async_copy(tabl_hbm.at[src_v1], rows_l, sem1).wait()
        pltpu.make_async_copy(tabr_hbm.at[dst_v1], rows_r, sem2).wait()

        plsc.subcore_barrier()
        pltpu.sync_copy(
            acc_sh.at[pl.ds(sid * rows_per_sub, rows_per_sub)],
            out_feat.at[cid, pl.ds(sid * rows_per_sub, rows_per_sub)],
        )

        @pl.when(sid == 0)
        def _write_esum():
            pltpu.sync_copy(accw_sh, out_w.at[cid])

    return edge_kernel


_edge_l1 = _make_edge_kernel(D1)
_edge_l2 = _edge_l1


def _tc1_body(x_ref, wl_ref, wr_ref, xl_ref, xr_ref):
    x = x_ref[...]
    xl_ref[...] = jnp.dot(x, wl_ref[...], preferred_element_type=jnp.float32)
    xr_ref[...] = jnp.dot(x, wr_ref[...], preferred_element_type=jnp.float32)


def _tc1(x, W1l, W1r):
    return pl.pallas_call(
        _tc1_body,
        grid=(NBLK,),
        in_specs=[
            pl.BlockSpec((BLK, D1), lambda i: (i, 0)),
            pl.BlockSpec((D1, D1), lambda i: (0, 0)),
            pl.BlockSpec((D1, D1), lambda i: (0, 0)),
        ],
        out_specs=[
            pl.BlockSpec((BLK, D1), lambda i: (i, 0)),
            pl.BlockSpec((BLK, D1), lambda i: (i, 0)),
        ],
        out_shape=[
            jax.ShapeDtypeStruct((ACC_N, D1), jnp.float32),
            jax.ShapeDtypeStruct((ACC_N, D1), jnp.float32),
        ],
    )(x, W1l, W1r)


def _tc2_body(acc_ref, es_ref, xl_ref, xr_ref, att_ref, b_ref,
              w2l_ref, w2r_ref, hl_ref, hr_ref):
    a = acc_ref[...]
    es = es_ref[...]
    xl = xl_ref[...]
    xr = xr_ref[...]
    s = xl + xr
    e = jnp.maximum(s, 0.2 * s)
    wself = jnp.exp(jnp.sum(e * att_ref[...], axis=-1, keepdims=True))
    numer = a[0] + a[1] + wself * xl
    denom = (es[0] + es[1])[:, None] + wself + EPS
    h = jnp.maximum(numer / denom + b_ref[...], 0.0)
    hl_ref[...] = jnp.dot(h, w2l_ref[...], preferred_element_type=jnp.float32)
    hr_ref[...] = jnp.dot(h, w2r_ref[...], preferred_element_type=jnp.float32)


def _tc2(acc1, es1, xl, xr, att1, b1, W2l_p, W2r_p):
    return pl.pallas_call(
        _tc2_body,
        grid=(NBLK,),
        in_specs=[
            pl.BlockSpec((NC, BLK, D1), lambda i: (0, i, 0)),
            pl.BlockSpec((NC, BLK), lambda i: (0, i)),
            pl.BlockSpec((BLK, D1), lambda i: (i, 0)),
            pl.BlockSpec((BLK, D1), lambda i: (i, 0)),
            pl.BlockSpec((1, D1), lambda i: (0, 0)),
            pl.BlockSpec((1, D1), lambda i: (0, 0)),
            pl.BlockSpec((D1, D1), lambda i: (0, 0)),
            pl.BlockSpec((D1, D1), lambda i: (0, 0)),
        ],
        out_specs=[
            pl.BlockSpec((BLK, D1), lambda i: (i, 0)),
            pl.BlockSpec((BLK, D1), lambda i: (i, 0)),
        ],
        out_shape=[
            jax.ShapeDtypeStruct((ACC_N, D1), jnp.float32),
            jax.ShapeDtypeStruct((ACC_N, D1), jnp.float32),
        ],
    )(acc1, es1, xl, xr, att1, b1, W2l_p, W2r_p)


def _tc3_body(acc_ref, es_ref, hl_ref, hr_ref, att_ref, b_ref, out_ref):
    a = acc_ref[...]
    es = es_ref[...]
    hl = hl_ref[...]
    hr = hr_ref[...]
    s = hl + hr
    e = jnp.maximum(s, 0.2 * s)
    wself = jnp.exp(jnp.sum(e * att_ref[...], axis=-1, keepdims=True))
    numer = a[0] + a[1] + wself * hl
    denom = (es[0] + es[1])[:, None] + wself + EPS
    out_ref[...] = (numer / denom)[:, :D2] + b_ref[...]


def _tc3(acc2, es2, hl, hr, att2p, b2):
    return pl.pallas_call(
        _tc3_body,
        grid=(NBLK,),
        in_specs=[
            pl.BlockSpec((NC, BLK, D1), lambda i: (0, i, 0)),
            pl.BlockSpec((NC, BLK), lambda i: (0, i)),
            pl.BlockSpec((BLK, D1), lambda i: (i, 0)),
            pl.BlockSpec((BLK, D1), lambda i: (i, 0)),
            pl.BlockSpec((1, D1), lambda i: (0, 0)),
            pl.BlockSpec((1, D2), lambda i: (0, 0)),
        ],
        out_specs=pl.BlockSpec((BLK, D2), lambda i: (i, 0)),
        out_shape=jax.ShapeDtypeStruct((ACC_N, D2), jnp.float32),
    )(acc2, es2, hl, hr, att2p, b2)


def kernel(x, edge_index, W1l, W1r, att1, b1, W2l, W2r, att2, b2):
    src = edge_index[0]
    dst = edge_index[1]
    x_pad = jnp.concatenate(
        [x, jnp.zeros((ACC_N - N, D1), jnp.float32)], axis=0
    )
    W2l_p = jnp.concatenate([W2l, jnp.zeros((D1, D1 - D2), jnp.float32)], 1)
    W2r_p = jnp.concatenate([W2r, jnp.zeros((D1, D1 - D2), jnp.float32)], 1)
    att2_p = jnp.concatenate(
        [att2.reshape(D2), jnp.zeros((D1 - D2,), jnp.float32)]
    )

    xl, xr = _tc1(x_pad, W1l, W1r)
    acc1, esw1 = _edge_l1(xl, xr, src, dst, att1.reshape(D1))
    es1 = esw1.reshape(NC, ACC_N)
    hl, hr = _tc2(acc1, es1, xl, xr, att1.reshape(1, D1), b1.reshape(1, D1),
                  W2l_p, W2r_p)
    acc2, esw2 = _edge_l2(hl, hr, src, dst, att2_p)
    es2 = esw2.reshape(NC, ACC_N)
    out = _tc3(acc2, es2, hl, hr, att2_p.reshape(1, D1), b2.reshape(1, D2))
    return out[:N]


# speedup vs baseline: 1.1655x; 1.1655x over previous
import functools

import jax
import jax.numpy as jnp
from jax import lax
from jax.experimental import pallas as pl
from jax.experimental.pallas import tpu as pltpu
from jax.experimental.pallas import tpu_sc as plsc

N = 10000
E = 320000
D1 = 128
D2 = 64
NC, NS = 2, 16
NW = NC * NS
EPW = E // NW
K = 80
NG = K // 16
NCH = EPW // K
ACC_N = 10240
WR = ACC_N // 128
BLK = 1024
NBLK = ACC_N // BLK
EPS = 1e-16

_GDN = lax.GatherDimensionNumbers(
    offset_dims=(), collapsed_slice_dims=(0,), start_index_map=(0,)
)


def _lane_perm(v, idx):
    return lax.gather(v, idx[:, None], _GDN, slice_sizes=(1,),
                      mode=lax.GatherScatterMode.PROMISE_IN_BOUNDS)


def _lane_sum_bcast(v):
    t = plsc.cumsum(v)
    return _lane_perm(t, jnp.full((16,), 15, jnp.int32))


def _make_edge_kernel(cga):
    D = D1
    cgn = D // 16
    rows_per_sub = ACC_N // NS
    mesh = plsc.VectorSubcoreMesh(
        core_axis_name="c", subcore_axis_name="s", num_cores=NC, num_subcores=NS
    )

    @functools.partial(
        pl.kernel,
        out_type=[
            jax.ShapeDtypeStruct((NC, ACC_N, D), jnp.float32),
            jax.ShapeDtypeStruct((NC, WR, 128), jnp.float32),
        ],
        mesh=mesh,
        scratch_types=[
            pltpu.VMEM((K,), jnp.int32),
            pltpu.VMEM((K,), jnp.int32),
            pltpu.VMEM((K,), jnp.int32),
            pltpu.VMEM((K,), jnp.int32),
            pltpu.VMEM((K,), jnp.int32),
            pltpu.VMEM((K,), jnp.int32),
            pltpu.VMEM((K, D), jnp.float32),
            pltpu.VMEM((K, D), jnp.float32),
            pltpu.VMEM((K, D), jnp.float32),
            pltpu.VMEM((K * 16,), jnp.float32),
            pltpu.VMEM((K, 128), jnp.float32),
            pltpu.VMEM((16 * cga,), jnp.float32),
            pltpu.VMEM_SHARED((ACC_N, D), jnp.float32),
            pltpu.VMEM_SHARED((WR, 128), jnp.float32),
            pltpu.SemaphoreType.DMA,
            pltpu.SemaphoreType.DMA,
            pltpu.SemaphoreType.DMA,
            pltpu.SemaphoreType.DMA,
            pltpu.SemaphoreType.DMA,
            pltpu.SemaphoreType.DMA,
        ],
        compiler_params=pltpu.CompilerParams(needs_layout_passes=False),
    )
    def edge_kernel(tabl_hbm, tabr_hbm, src_hbm, dst_hbm, att_hbm,
                    out_feat, out_w,
                    src_v0, dst_v0, ddiv_v0, src_v1, dst_v1, ddiv_v1,
                    rows_l, rows_r, packed, wbuf,
                    wrows, att_v, acc_sh, accw_sh,
                    sem1, sem2, sems1, sems2, semidx1, semidx2):
        cid = lax.axis_index("c")
        sid = lax.axis_index("s")
        wid = sid * NC + cid
        zero16 = jnp.zeros((16,), jnp.float32)
        iota16 = lax.iota(jnp.int32, 16)

        def zp(i, carry):
            for j in range(cgn):
                packed[i, pl.ds(j * 16, 16)] = zero16
            for j in range(8):
                wrows[i, pl.ds(j * 16, 16)] = zero16
            return carry
        lax.fori_loop(0, K, zp, 0)

        for t in range(rows_per_sub // K):
            pltpu.sync_copy(
                packed, acc_sh.at[pl.ds(sid * rows_per_sub + t * K, K)]
            )

        @pl.when(sid == 0)
        def _zero_esum():
            pltpu.sync_copy(wrows, accw_sh)

        pltpu.sync_copy(att_hbm, att_v)
        att_regs = [att_v[pl.ds(j * 16, 16)] for j in range(cga)]
        plsc.subcore_barrier()

        idx_bufs = [(src_v0, dst_v0, ddiv_v0), (src_v1, dst_v1, ddiv_v1)]

        def do_chunk(k, cur, other, first):
            srcb, dstb, ddivb = cur
            psrc, pdst, pddiv = other
            if not first:
                pltpu.make_async_copy(packed, acc_sh.at[pdst], sems1).wait()
                pltpu.make_async_copy(wrows, accw_sh.at[pddiv], sems2).wait()
                for g in range(NG):
                    eids = g * 16 + iota16
                    dsts = pdst[pl.ds(g * 16, 16)]
                    dmod = lax.bitwise_and(dsts, 127)
                    plsc.store_scatter(wrows, [eids, dmod], zero16)
            nbase = wid * EPW + jnp.minimum(k + 1, NCH - 1) * K
            pltpu.async_copy(src_hbm.at[pl.ds(nbase, K)], psrc, semidx1)
            pltpu.async_copy(dst_hbm.at[pl.ds(nbase, K)], pdst, semidx2)
            pltpu.make_async_copy(tabl_hbm.at[srcb], rows_l, sem1).wait()
            pltpu.make_async_copy(tabr_hbm.at[dstb], rows_r, sem2).wait()

            @plsc.parallel_loop(0, K, unroll=4)
            def edge_body(e):
                ls = []
                acc = None
                for j in range(cga):
                    l = rows_l[e, pl.ds(j * 16, 16)]
                    r = rows_r[e, pl.ds(j * 16, 16)]
                    ls.append(l)
                    s = l + r
                    lr = jnp.maximum(s, 0.2 * s)
                    term = lr * att_regs[j]
                    acc = term if acc is None else acc + term
                w = jnp.exp(_lane_sum_bcast(acc))
                for j in range(cga):
                    packed[e, pl.ds(j * 16, 16)] = ls[j] * w
                wbuf[pl.ds(e * 16, 16)] = w

            for g in range(NG):
                eids = g * 16 + iota16
                dsts = dstb[pl.ds(g * 16, 16)]
                dmod = lax.bitwise_and(dsts, 127)
                ddiv = lax.shift_right_logical(dsts, 7)
                ddivb[pl.ds(g * 16, 16)] = ddiv
                wv = plsc.load_gather(wbuf, [eids * 16])
                plsc.store_scatter(wrows, [eids, dmod], wv)

            pltpu.async_copy(packed, acc_sh.at[dstb], sems1, add=True)
            pltpu.async_copy(wrows, accw_sh.at[ddivb], sems2, add=True)
            pltpu.make_async_copy(
                src_hbm.at[pl.ds(0, K)], psrc, semidx1).wait()
            pltpu.make_async_copy(
                dst_hbm.at[pl.ds(0, K)], pdst, semidx2).wait()
            pltpu.async_copy(tabl_hbm.at[psrc], rows_l, sem1)
            pltpu.async_copy(tabr_hbm.at[pdst], rows_r, sem2)

        base0 = wid * EPW
        pltpu.sync_copy(src_hbm.at[pl.ds(base0, K)], src_v0)
        pltpu.sync_copy(dst_hbm.at[pl.ds(base0, K)], dst_v0)
        pltpu.async_copy(tabl_hbm.at[src_v0], rows_l, sem1)
        pltpu.async_copy(tabr_hbm.at[dst_v0], rows_r, sem2)
        do_chunk(0, idx_bufs[0], idx_bufs[1], True)

        def pair_body(i, carry):
            do_chunk(1 + 2 * i, idx_bufs[1], idx_bufs[0], False)
            do_chunk(2 + 2 * i, idx_bufs[0], idx_bufs[1], False)
            return carry
        lax.fori_loop(0, (NCH - 1) // 2, pair_body, 0)

        pltpu.make_async_copy(packed, acc_sh.at[dst_v0], sems1).wait()
        pltpu.make_async_copy(wrows, accw_sh.at[ddiv_v0], sems2).wait()
        pltpu.make_async_copy(tabl_hbm.at[src_v1], rows_l, sem1).wait()
        pltpu.make_async_copy(tabr_hbm.at[dst_v1], rows_r, sem2).wait()

        plsc.subcore_barrier()
        pltpu.sync_copy(
            acc_sh.at[pl.ds(sid * rows_per_sub, rows_per_sub)],
            out_feat.at[cid, pl.ds(sid * rows_per_sub, rows_per_sub)],
        )

        @pl.when(sid == 0)
        def _write_esum():
            pltpu.sync_copy(accw_sh, out_w.at[cid])

    return edge_kernel


_edge_l1 = _make_edge_kernel(8)
_edge_l2 = _make_edge_kernel(4)


def _tc1_body(x_ref, wl_ref, wr_ref, xl_ref, xr_ref):
    x = x_ref[...]
    xl_ref[...] = jnp.dot(x, wl_ref[...], preferred_element_type=jnp.float32)
    xr_ref[...] = jnp.dot(x, wr_ref[...], preferred_element_type=jnp.float32)


def _tc1(x, W1l, W1r):
    return pl.pallas_call(
        _tc1_body,
        grid=(NBLK,),
        in_specs=[
            pl.BlockSpec((BLK, D1), lambda i: (i, 0)),
            pl.BlockSpec((D1, D1), lambda i: (0, 0)),
            pl.BlockSpec((D1, D1), lambda i: (0, 0)),
        ],
        out_specs=[
            pl.BlockSpec((BLK, D1), lambda i: (i, 0)),
            pl.BlockSpec((BLK, D1), lambda i: (i, 0)),
        ],
        out_shape=[
            jax.ShapeDtypeStruct((ACC_N, D1), jnp.float32),
            jax.ShapeDtypeStruct((ACC_N, D1), jnp.float32),
        ],
    )(x, W1l, W1r)


def _tc2_body(acc_ref, es_ref, xl_ref, xr_ref, att_ref, b_ref,
              w2l_ref, w2r_ref, hl_ref, hr_ref):
    a = acc_ref[...]
    es = es_ref[...]
    xl = xl_ref[...]
    xr = xr_ref[...]
    s = xl + xr
    e = jnp.maximum(s, 0.2 * s)
    wself = jnp.exp(jnp.sum(e * att_ref[...], axis=-1, keepdims=True))
    numer = a[0] + a[1] + wself * xl
    denom = (es[0] + es[1])[:, None] + wself + EPS
    h = jnp.maximum(numer / denom + b_ref[...], 0.0)
    hl_ref[...] = jnp.dot(h, w2l_ref[...], preferred_element_type=jnp.float32)
    hr_ref[...] = jnp.dot(h, w2r_ref[...], preferred_element_type=jnp.float32)


def _tc2(acc1, es1, xl, xr, att1, b1, W2l_p, W2r_p):
    return pl.pallas_call(
        _tc2_body,
        grid=(NBLK,),
        in_specs=[
            pl.BlockSpec((NC, BLK, D1), lambda i: (0, i, 0)),
            pl.BlockSpec((NC, BLK), lambda i: (0, i)),
            pl.BlockSpec((BLK, D1), lambda i: (i, 0)),
            pl.BlockSpec((BLK, D1), lambda i: (i, 0)),
            pl.BlockSpec((1, D1), lambda i: (0, 0)),
            pl.BlockSpec((1, D1), lambda i: (0, 0)),
            pl.BlockSpec((D1, D1), lambda i: (0, 0)),
            pl.BlockSpec((D1, D1), lambda i: (0, 0)),
        ],
        out_specs=[
            pl.BlockSpec((BLK, D1), lambda i: (i, 0)),
            pl.BlockSpec((BLK, D1), lambda i: (i, 0)),
        ],
        out_shape=[
            jax.ShapeDtypeStruct((ACC_N, D1), jnp.float32),
            jax.ShapeDtypeStruct((ACC_N, D1), jnp.float32),
        ],
    )(acc1, es1, xl, xr, att1, b1, W2l_p, W2r_p)


def _tc3_body(acc_ref, es_ref, hl_ref, hr_ref, att_ref, b_ref, out_ref):
    a = acc_ref[...]
    es = es_ref[...]
    hl = hl_ref[...]
    hr = hr_ref[...]
    s = hl + hr
    e = jnp.maximum(s, 0.2 * s)
    wself = jnp.exp(jnp.sum(e * att_ref[...], axis=-1, keepdims=True))
    numer = a[0] + a[1] + wself * hl
    denom = (es[0] + es[1])[:, None] + wself + EPS
    out_ref[...] = (numer / denom)[:, :D2] + b_ref[...]


def _tc3(acc2, es2, hl, hr, att2p, b2):
    return pl.pallas_call(
        _tc3_body,
        grid=(NBLK,),
        in_specs=[
            pl.BlockSpec((NC, BLK, D1), lambda i: (0, i, 0)),
            pl.BlockSpec((NC, BLK), lambda i: (0, i)),
            pl.BlockSpec((BLK, D1), lambda i: (i, 0)),
            pl.BlockSpec((BLK, D1), lambda i: (i, 0)),
            pl.BlockSpec((1, D1), lambda i: (0, 0)),
            pl.BlockSpec((1, D2), lambda i: (0, 0)),
        ],
        out_specs=pl.BlockSpec((BLK, D2), lambda i: (i, 0)),
        out_shape=jax.ShapeDtypeStruct((ACC_N, D2), jnp.float32),
    )(acc2, es2, hl, hr, att2p, b2)


def kernel(x, edge_index, W1l, W1r, att1, b1, W2l, W2r, att2, b2):
    src = edge_index[0]
    dst = edge_index[1]
    x_pad = jnp.concatenate(
        [x, jnp.zeros((ACC_N - N, D1), jnp.float32)], axis=0
    )
    W2l_p = jnp.concatenate([W2l, jnp.zeros((D1, D1 - D2), jnp.float32)], 1)
    W2r_p = jnp.concatenate([W2r, jnp.zeros((D1, D1 - D2), jnp.float32)], 1)
    att2_p = jnp.concatenate(
        [att2.reshape(D2), jnp.zeros((D1 - D2,), jnp.float32)]
    )

    xl, xr = _tc1(x_pad, W1l, W1r)
    acc1, esw1 = _edge_l1(xl, xr, src, dst, att1.reshape(D1))
    es1 = esw1.reshape(NC, ACC_N)
    hl, hr = _tc2(acc1, es1, xl, xr, att1.reshape(1, D1), b1.reshape(1, D1),
                  W2l_p, W2r_p)
    acc2, esw2 = _edge_l2(hl, hr, src, dst, att2.reshape(D2))
    es2 = esw2.reshape(NC, ACC_N)
    out = _tc3(acc2, es2, hl, hr, att2_p.reshape(1, D1), b2.reshape(1, D2))
    return out[:N]


# speedup vs baseline: 1.2333x; 1.0582x over previous
import functools

import jax
import jax.numpy as jnp
from jax import lax
from jax.experimental import pallas as pl
from jax.experimental.pallas import tpu as pltpu
from jax.experimental.pallas import tpu_sc as plsc

N = 10000
E = 320000
D1 = 128
D2 = 64
NC, NS = 2, 16
NW = NC * NS
EPW = E // NW
K = 80
NG = K // 16
NCH = EPW // K
ACC_N = 10240
WR = ACC_N // 128
BLK = 1024
NBLK = ACC_N // BLK
EPS = 1e-16

_GDN = lax.GatherDimensionNumbers(
    offset_dims=(), collapsed_slice_dims=(0,), start_index_map=(0,)
)


def _lane_perm(v, idx):
    return lax.gather(v, idx[:, None], _GDN, slice_sizes=(1,),
                      mode=lax.GatherScatterMode.PROMISE_IN_BOUNDS)


def _lane_sum_bcast(v):
    t = plsc.cumsum(v)
    return _lane_perm(t, jnp.full((16,), 15, jnp.int32))


def _make_edge_kernel(cga, fold_esum):
    D = D1
    cgn = D // 16
    rows_per_sub = ACC_N // NS
    mesh = plsc.VectorSubcoreMesh(
        core_axis_name="c", subcore_axis_name="s", num_cores=NC, num_subcores=NS
    )

    out_types = [jax.ShapeDtypeStruct((NC, ACC_N, D), jnp.float32)]
    if not fold_esum:
        out_types.append(jax.ShapeDtypeStruct((NC, WR, 128), jnp.float32))
    scratch = [
        pltpu.VMEM((K,), jnp.int32),
        pltpu.VMEM((K,), jnp.int32),
        pltpu.VMEM((K,), jnp.int32),
        pltpu.VMEM((K,), jnp.int32),
        pltpu.VMEM((K, D), jnp.float32),
        pltpu.VMEM((K, D), jnp.float32),
        pltpu.VMEM((K, D), jnp.float32),
        pltpu.VMEM((16 * cga,), jnp.float32),
    ]
    if not fold_esum:
        scratch += [
            pltpu.VMEM((K,), jnp.int32),
            pltpu.VMEM((K,), jnp.int32),
            pltpu.VMEM((K * 16,), jnp.float32),
            pltpu.VMEM((K, 128), jnp.float32),
        ]
    scratch.append(pltpu.VMEM_SHARED((ACC_N, D), jnp.float32))
    if not fold_esum:
        scratch.append(pltpu.VMEM_SHARED((WR, 128), jnp.float32))
    nsem = 5 if fold_esum else 6
    scratch += [pltpu.SemaphoreType.DMA] * nsem

    @functools.partial(
        pl.kernel,
        out_type=out_types,
        mesh=mesh,
        scratch_types=scratch,
        compiler_params=pltpu.CompilerParams(needs_layout_passes=False),
    )
    def edge_kernel(tabl_hbm, tabr_hbm, src_hbm, dst_hbm, att_hbm, *rest):
        it = iter(rest)
        out_feat = next(it)
        out_w = None if fold_esum else next(it)
        src_v0, dst_v0, src_v1, dst_v1 = (next(it) for _ in range(4))
        rows_l, rows_r, packed, att_v = (next(it) for _ in range(4))
        if fold_esum:
            ddiv_v0 = ddiv_v1 = wbuf = wrows = accw_sh = None
        else:
            ddiv_v0, ddiv_v1, wbuf, wrows = (next(it) for _ in range(4))
        acc_sh = next(it)
        if not fold_esum:
            accw_sh = next(it)
        sem1, sem2, sems1, semidx1, semidx2 = (next(it) for _ in range(5))
        sems2 = None if fold_esum else next(it)

        cid = lax.axis_index("c")
        sid = lax.axis_index("s")
        wid = sid * NC + cid
        zero16 = jnp.zeros((16,), jnp.float32)
        iota16 = lax.iota(jnp.int32, 16)
        m0f = (iota16 == 0).astype(jnp.float32)

        def zp(i, carry):
            for j in range(cgn):
                packed[i, pl.ds(j * 16, 16)] = zero16
            if not fold_esum:
                for j in range(8):
                    wrows[i, pl.ds(j * 16, 16)] = zero16
            return carry
        lax.fori_loop(0, K, zp, 0)

        for t in range(rows_per_sub // K):
            pltpu.sync_copy(
                packed, acc_sh.at[pl.ds(sid * rows_per_sub + t * K, K)]
            )

        if not fold_esum:
            @pl.when(sid == 0)
            def _zero_esum():
                pltpu.sync_copy(wrows, accw_sh)

        pltpu.sync_copy(att_hbm, att_v)
        att_regs = [att_v[pl.ds(j * 16, 16)] for j in range(cga)]
        plsc.subcore_barrier()

        if fold_esum:
            idx_bufs = [(src_v0, dst_v0, None), (src_v1, dst_v1, None)]
        else:
            idx_bufs = [(src_v0, dst_v0, ddiv_v0), (src_v1, dst_v1, ddiv_v1)]

        def do_chunk(k, cur, other, first):
            srcb, dstb, ddivb = cur
            psrc, pdst, pddiv = other
            if not first:
                pltpu.make_async_copy(packed, acc_sh.at[pdst], sems1).wait()
                if not fold_esum:
                    pltpu.make_async_copy(
                        wrows, accw_sh.at[pddiv], sems2).wait()
                    for g in range(NG):
                        eids = g * 16 + iota16
                        dsts = pdst[pl.ds(g * 16, 16)]
                        dmod = lax.bitwise_and(dsts, 127)
                        plsc.store_scatter(wrows, [eids, dmod], zero16)
            nbase = wid * EPW + jnp.minimum(k + 1, NCH - 1) * K
            pltpu.async_copy(src_hbm.at[pl.ds(nbase, K)], psrc, semidx1)
            pltpu.async_copy(dst_hbm.at[pl.ds(nbase, K)], pdst, semidx2)
            pltpu.make_async_copy(tabl_hbm.at[srcb], rows_l, sem1).wait()
            pltpu.make_async_copy(tabr_hbm.at[dstb], rows_r, sem2).wait()

            @plsc.parallel_loop(0, K, unroll=4)
            def edge_body(e):
                ls = []
                acc = None
                for j in range(cga):
                    l = rows_l[e, pl.ds(j * 16, 16)]
                    r = rows_r[e, pl.ds(j * 16, 16)]
                    ls.append(l)
                    s = l + r
                    lr = jnp.maximum(s, 0.2 * s)
                    term = lr * att_regs[j]
                    acc = term if acc is None else acc + term
                w = jnp.exp(_lane_sum_bcast(acc))
                for j in range(cga):
                    packed[e, pl.ds(j * 16, 16)] = ls[j] * w
                if fold_esum:
                    packed[e, pl.ds(16 * cga, 16)] = w * m0f
                else:
                    wbuf[pl.ds(e * 16, 16)] = w

            if not fold_esum:
                for g in range(NG):
                    eids = g * 16 + iota16
                    dsts = dstb[pl.ds(g * 16, 16)]
                    dmod = lax.bitwise_and(dsts, 127)
                    ddiv = lax.shift_right_logical(dsts, 7)
                    ddivb[pl.ds(g * 16, 16)] = ddiv
                    wv = plsc.load_gather(wbuf, [eids * 16])
                    plsc.store_scatter(wrows, [eids, dmod], wv)

            pltpu.async_copy(packed, acc_sh.at[dstb], sems1, add=True)
            if not fold_esum:
                pltpu.async_copy(wrows, accw_sh.at[ddivb], sems2, add=True)
            pltpu.make_async_copy(
                src_hbm.at[pl.ds(0, K)], psrc, semidx1).wait()
            pltpu.make_async_copy(
                dst_hbm.at[pl.ds(0, K)], pdst, semidx2).wait()
            pltpu.async_copy(tabl_hbm.at[psrc], rows_l, sem1)
            pltpu.async_copy(tabr_hbm.at[pdst], rows_r, sem2)

        base0 = wid * EPW
        pltpu.sync_copy(src_hbm.at[pl.ds(base0, K)], src_v0)
        pltpu.sync_copy(dst_hbm.at[pl.ds(base0, K)], dst_v0)
        pltpu.async_copy(tabl_hbm.at[src_v0], rows_l, sem1)
        pltpu.async_copy(tabr_hbm.at[dst_v0], rows_r, sem2)
        do_chunk(0, idx_bufs[0], idx_bufs[1], True)

        def pair_body(i, carry):
            do_chunk(1 + 2 * i, idx_bufs[1], idx_bufs[0], False)
            do_chunk(2 + 2 * i, idx_bufs[0], idx_bufs[1], False)
            return carry
        lax.fori_loop(0, (NCH - 1) // 2, pair_body, 0)

        pltpu.make_async_copy(packed, acc_sh.at[dst_v0], sems1).wait()
        if not fold_esum:
            pltpu.make_async_copy(wrows, accw_sh.at[ddiv_v0], sems2).wait()
        pltpu.make_async_copy(tabl_hbm.at[src_v1], rows_l, sem1).wait()
        pltpu.make_async_copy(tabr_hbm.at[dst_v1], rows_r, sem2).wait()

        plsc.subcore_barrier()
        pltpu.sync_copy(
            acc_sh.at[pl.ds(sid * rows_per_sub, rows_per_sub)],
            out_feat.at[cid, pl.ds(sid * rows_per_sub, rows_per_sub)],
        )

        if not fold_esum:
            @pl.when(sid == 0)
            def _write_esum():
                pltpu.sync_copy(accw_sh, out_w.at[cid])

    return edge_kernel


_edge_l1 = _make_edge_kernel(8, fold_esum=False)
_edge_l2 = _make_edge_kernel(4, fold_esum=True)


def _tc1_body(x_ref, wl_ref, wr_ref, xl_ref, xr_ref):
    x = x_ref[...]
    xl_ref[...] = jnp.dot(x, wl_ref[...], preferred_element_type=jnp.float32)
    xr_ref[...] = jnp.dot(x, wr_ref[...], preferred_element_type=jnp.float32)


def _tc1(x, W1l, W1r):
    return pl.pallas_call(
        _tc1_body,
        grid=(NBLK,),
        in_specs=[
            pl.BlockSpec((BLK, D1), lambda i: (i, 0)),
            pl.BlockSpec((D1, D1), lambda i: (0, 0)),
            pl.BlockSpec((D1, D1), lambda i: (0, 0)),
        ],
        out_specs=[
            pl.BlockSpec((BLK, D1), lambda i: (i, 0)),
            pl.BlockSpec((BLK, D1), lambda i: (i, 0)),
        ],
        out_shape=[
            jax.ShapeDtypeStruct((ACC_N, D1), jnp.float32),
            jax.ShapeDtypeStruct((ACC_N, D1), jnp.float32),
        ],
    )(x, W1l, W1r)


def _tc2_body(acc_ref, es_ref, xl_ref, xr_ref, att_ref, b_ref,
              w2l_ref, w2r_ref, hl_ref, hr_ref):
    a = acc_ref[...]
    es = es_ref[...]
    xl = xl_ref[...]
    xr = xr_ref[...]
    s = xl + xr
    e = jnp.maximum(s, 0.2 * s)
    wself = jnp.exp(jnp.sum(e * att_ref[...], axis=-1, keepdims=True))
    numer = a[0] + a[1] + wself * xl
    denom = (es[0] + es[1])[:, None] + wself + EPS
    h = jnp.maximum(numer / denom + b_ref[...], 0.0)
    hl_ref[...] = jnp.dot(h, w2l_ref[...], preferred_element_type=jnp.float32)
    hr_ref[...] = jnp.dot(h, w2r_ref[...], preferred_element_type=jnp.float32)


def _tc2(acc1, es1, xl, xr, att1, b1, W2l_p, W2r_p):
    return pl.pallas_call(
        _tc2_body,
        grid=(NBLK,),
        in_specs=[
            pl.BlockSpec((NC, BLK, D1), lambda i: (0, i, 0)),
            pl.BlockSpec((NC, BLK), lambda i: (0, i)),
            pl.BlockSpec((BLK, D1), lambda i: (i, 0)),
            pl.BlockSpec((BLK, D1), lambda i: (i, 0)),
            pl.BlockSpec((1, D1), lambda i: (0, 0)),
            pl.BlockSpec((1, D1), lambda i: (0, 0)),
            pl.BlockSpec((D1, D1), lambda i: (0, 0)),
            pl.BlockSpec((D1, D1), lambda i: (0, 0)),
        ],
        out_specs=[
            pl.BlockSpec((BLK, D1), lambda i: (i, 0)),
            pl.BlockSpec((BLK, D1), lambda i: (i, 0)),
        ],
        out_shape=[
            jax.ShapeDtypeStruct((ACC_N, D1), jnp.float32),
            jax.ShapeDtypeStruct((ACC_N, D1), jnp.float32),
        ],
    )(acc1, es1, xl, xr, att1, b1, W2l_p, W2r_p)


def _tc3_body(acc_ref, hl_ref, hr_ref, att_ref, b_ref, out_ref):
    a = acc_ref[...]
    hl = hl_ref[...]
    hr = hr_ref[...]
    s = hl + hr
    e = jnp.maximum(s, 0.2 * s)
    wself = jnp.exp(jnp.sum(e * att_ref[...], axis=-1, keepdims=True))
    asum = a[0] + a[1]
    numer = asum[:, :D2] + wself * hl[:, :D2]
    denom = asum[:, D2:D2 + 1] + wself + EPS
    out_ref[...] = numer / denom + b_ref[...]


def _tc3(acc2, hl, hr, att2p, b2):
    return pl.pallas_call(
        _tc3_body,
        grid=(NBLK,),
        in_specs=[
            pl.BlockSpec((NC, BLK, D1), lambda i: (0, i, 0)),
            pl.BlockSpec((BLK, D1), lambda i: (i, 0)),
            pl.BlockSpec((BLK, D1), lambda i: (i, 0)),
            pl.BlockSpec((1, D1), lambda i: (0, 0)),
            pl.BlockSpec((1, D2), lambda i: (0, 0)),
        ],
        out_specs=pl.BlockSpec((BLK, D2), lambda i: (i, 0)),
        out_shape=jax.ShapeDtypeStruct((ACC_N, D2), jnp.float32),
    )(acc2, hl, hr, att2p, b2)


def kernel(x, edge_index, W1l, W1r, att1, b1, W2l, W2r, att2, b2):
    src = edge_index[0]
    dst = edge_index[1]
    x_pad = jnp.concatenate(
        [x, jnp.zeros((ACC_N - N, D1), jnp.float32)], axis=0
    )
    W2l_p = jnp.concatenate([W2l, jnp.zeros((D1, D1 - D2), jnp.float32)], 1)
    W2r_p = jnp.concatenate([W2r, jnp.zeros((D1, D1 - D2), jnp.float32)], 1)
    att2_p = jnp.concatenate(
        [att2.reshape(D2), jnp.zeros((D1 - D2,), jnp.float32)]
    )

    xl, xr = _tc1(x_pad, W1l, W1r)
    acc1, esw1 = _edge_l1(xl, xr, src, dst, att1.reshape(D1))
    es1 = esw1.reshape(NC, ACC_N)
    hl, hr = _tc2(acc1, es1, xl, xr, att1.reshape(1, D1), b1.reshape(1, D1),
                  W2l_p, W2r_p)
    acc2, = _edge_l2(hl, hr, src, dst, att2.reshape(D2))
    out = _tc3(acc2, hl, hr, att2_p.reshape(1, D1), b2.reshape(1, D2))
    return out[:N]
